# Initial kernel scaffold; baseline (speedup 1.0000x reference)
import jax
import jax.numpy as jnp
from jax.experimental import pallas as pl


def kernel(x, Wg, W1, b1, W2, b2, Wv, bv, Wout, bout, bk_scale):
    raise NotImplementedError("write your pallas kernel here")


# speedup vs baseline: 28.7501x; 28.7501x over previous
import functools

import jax
import jax.numpy as jnp
from jax.experimental import pallas as pl
from jax.experimental.pallas import tpu as pltpu

D_MODEL = 768
N_SEQ = 2048
E = 8
TOP_K = 2
D_FF = 3072
V_MAX = 3.0
FEATURE_CLAMP = 10.0

TN = 1024
TF = 768
NT = N_SEQ // TN
NF = D_FF // TF


def _moe_body(x_ref, wg_ref, w1_ref, b1_ref, w2_ref, b2_ref, wv_ref, bv_ref,
              ffn_ref, v_ref, gates_s, acc_s):
    e = pl.program_id(1)
    f = pl.program_id(2)

    @pl.when((e == 0) & (f == 0))
    def _router():
        logits = jnp.dot(x_ref[...], wg_ref[...],
                         preferred_element_type=jnp.float32)
        m = jnp.max(logits, axis=-1, keepdims=True)
        ex = jnp.exp(logits - m)
        probs = ex / jnp.sum(ex, axis=-1, keepdims=True)
        iota = jax.lax.broadcasted_iota(jnp.int32, probs.shape, 1)
        p1 = jnp.max(probs, axis=-1, keepdims=True)
        i1 = jnp.min(jnp.where(probs == p1, iota, E), axis=-1, keepdims=True)
        hot1 = iota == i1
        masked = jnp.where(hot1, -jnp.inf, probs)
        p2 = jnp.max(masked, axis=-1, keepdims=True)
        i2 = jnp.min(jnp.where(masked == p2, iota, E), axis=-1, keepdims=True)
        hot2 = iota == i2
        denom = p1 + p2 + 1e-9
        gates_s[...] = jnp.where(hot1, p1 / denom,
                                 jnp.where(hot2, p2 / denom, 0.0))
        acc_s[...] = jnp.zeros_like(acc_s)

    h = jnp.maximum(
        jnp.dot(x_ref[...], w1_ref[0], preferred_element_type=jnp.float32)
        + b1_ref[0, 0], 0.0)
    y = jnp.dot(h, w2_ref[0], preferred_element_type=jnp.float32)
    g_all = gates_s[...]
    lane = jax.lax.broadcasted_iota(jnp.int32, g_all.shape, 1)
    gate = jnp.sum(jnp.where(lane == e, g_all, 0.0), axis=1, keepdims=True)
    acc_s[...] += gate * y

    @pl.when((e == E - 1) & (f == NF - 1))
    def _finish():
        bias2 = jnp.dot(gates_s[...], b2_ref[...],
                        preferred_element_type=jnp.float32)
        ffn = acc_s[...] + bias2
        ffn_ref[...] = ffn
        vt = jnp.dot(ffn, wv_ref[...],
                     preferred_element_type=jnp.float32) + bv_ref[0, 0]
        v_ref[...] = jnp.clip(vt, -V_MAX, V_MAX)


def _moe(xt, Wg, W1, b1, W2, b2, Wv, bv2):
    grid = (NT, E, NF)
    ffn, v = pl.pallas_call(
        _moe_body,
        grid=grid,
        in_specs=[
            pl.BlockSpec((TN, D_MODEL), lambda t, e, f: (t, 0)),
            pl.BlockSpec((D_MODEL, E), lambda t, e, f: (0, 0)),
            pl.BlockSpec((1, D_MODEL, TF), lambda t, e, f: (e, 0, f)),
            pl.BlockSpec((1, 1, TF), lambda t, e, f: (e, 0, f)),
            pl.BlockSpec((1, TF, D_MODEL), lambda t, e, f: (e, f, 0)),
            pl.BlockSpec((E, D_MODEL), lambda t, e, f: (0, 0)),
            pl.BlockSpec((D_MODEL, 1), lambda t, e, f: (0, 0)),
            pl.BlockSpec((1, 1), lambda t, e, f: (0, 0)),
        ],
        out_specs=[
            pl.BlockSpec((TN, D_MODEL), lambda t, e, f: (t, 0)),
            pl.BlockSpec((TN, 1), lambda t, e, f: (t, 0)),
        ],
        out_shape=[
            jax.ShapeDtypeStruct((N_SEQ, D_MODEL), jnp.float32),
            jax.ShapeDtypeStruct((N_SEQ, 1), jnp.float32),
        ],
        scratch_shapes=[
            pltpu.VMEM((TN, E), jnp.float32),
            pltpu.VMEM((TN, D_MODEL), jnp.float32),
        ],
    )(xt, Wg, W1, b1, W2, b2, Wv, bv2)
    return ffn, v


def _cmul(xr, xi, yr, yi):
    return xr * yr - xi * yi, xr * yi + xi * yr


def _matmul2(L, Ech):
    la_r, la_i, lb_r, lb_i, lc_r, lc_i, ld_r, ld_i = L
    ea_r, ea_i, eb_r, eb_i, ec_r, ec_i, ed_r, ed_i = Ech
    t1r, t1i = _cmul(la_r, la_i, ea_r, ea_i)
    t2r, t2i = _cmul(lb_r, lb_i, ec_r, ec_i)
    pa_r, pa_i = t1r + t2r, t1i + t2i
    t1r, t1i = _cmul(la_r, la_i, eb_r, eb_i)
    t2r, t2i = _cmul(lb_r, lb_i, ed_r, ed_i)
    pb_r, pb_i = t1r + t2r, t1i + t2i
    t1r, t1i = _cmul(lc_r, lc_i, ea_r, ea_i)
    t2r, t2i = _cmul(ld_r, ld_i, ec_r, ec_i)
    pc_r, pc_i = t1r + t2r, t1i + t2i
    t1r, t1i = _cmul(lc_r, lc_i, eb_r, eb_i)
    t2r, t2i = _cmul(ld_r, ld_i, ed_r, ed_i)
    pd_r, pd_i = t1r + t2r, t1i + t2i
    return (pa_r, pa_i, pb_r, pb_i, pc_r, pc_i, pd_r, pd_i)


_ID = (1.0, 0.0, 0.0, 0.0, 0.0, 0.0, 1.0, 0.0)


def _normalize(M):
    m = jnp.abs(M[0])
    for ch in M[1:]:
        m = jnp.maximum(m, jnp.abs(ch))
    inv = 1.0 / m
    return tuple(ch * inv for ch in M)


def _mobius_scan(M, n, forward):
    s = 1
    while s < n:
        shifted = []
        for ch, idv in zip(M, _ID):
            fill = jnp.full((1, s), idv, dtype=jnp.float32)
            if forward:
                sh = jnp.concatenate([fill, ch[:, : n - s]], axis=1)
            else:
                sh = jnp.concatenate([ch[:, s:], fill], axis=1)
            shifted.append(sh)
        M = _normalize(_matmul2(M, tuple(shifted)))
        s *= 2
    return M


def _bk_body(v_ref, g_ref):
    v = v_ref[...]
    d_re = 2.0 - v
    d_im = jnp.ones_like(v)
    zero = jnp.zeros_like(v)
    one = jnp.ones_like(v)
    M0 = (d_re, d_im, -one, zero, one, zero, zero, zero)

    PL = _mobius_scan(M0, N_SEQ, forward=True)
    PR = _mobius_scan(M0, N_SEQ, forward=False)

    def col_ratio(P):
        ar, ai, _, _, cr, ci, _, _ = P
        den = cr * cr + ci * ci
        return (ar * cr + ai * ci) / den, (ai * cr - ar * ci) / den

    l_re, l_im = col_ratio(PL)
    r_re, r_im = col_ratio(PR)
    den_re = l_re + r_re - d_re
    den_im = l_im + r_im - d_im
    mag = den_re * den_re + den_im * den_im
    g_re = den_re / mag
    g_im = -den_im / mag
    g_ref[0:1, :] = jnp.clip(g_re, -FEATURE_CLAMP, FEATURE_CLAMP)
    g_ref[1:2, :] = jnp.clip(g_im, -FEATURE_CLAMP, FEATURE_CLAMP)


def _bk(v_row):
    return pl.pallas_call(
        _bk_body,
        out_shape=jax.ShapeDtypeStruct((2, N_SEQ), jnp.float32),
    )(v_row)


def _combine_body(ffn_ref, f0_ref, f1_ref, wout_ref, bout_ref, bk_ref, o_ref):
    spec = (f0_ref[...] * wout_ref[0:1, :]
            + f1_ref[...] * wout_ref[1:2, :] + bout_ref[...])
    o_ref[...] = ffn_ref[...] + bk_ref[0, 0] * spec


def _combine(ffn, f0, f1, Wout, bout2, bk2):
    return pl.pallas_call(
        _combine_body,
        out_shape=jax.ShapeDtypeStruct((N_SEQ, D_MODEL), jnp.float32),
    )(ffn, f0, f1, Wout, bout2, bk2)


def kernel(x, Wg, W1, b1, W2, b2, Wv, bv, Wout, bout, bk_scale):
    B, N, D = x.shape
    xt = x.reshape(N, D)
    bv2 = bv.reshape(1, 1)
    ffn, v = _moe(xt, Wg, W1, b1.reshape(E, 1, D_FF), W2, b2, Wv, bv2)
    g = _bk(v.reshape(1, N))
    f0 = g[0].reshape(N, 1)
    f1 = g[1].reshape(N, 1)
    out = _combine(ffn, f0, f1, Wout, bout.reshape(1, D),
                   bk_scale.reshape(1, 1))
    return out.reshape(B, N, D)
